# baseline (speedup 1.0000x reference)
import dataclasses
import functools

import jax
import jax.numpy as jnp
from jax import lax
from jax.experimental import pallas as pl
from jax.experimental.pallas import tpu as pltpu
from jax.experimental.pallas import tpu_sc as plsc

N_NODES = 10000
N_EDGES = 320000
DIM = 128
N_GRAPHS = 64
EDGE_BLK = 128
NUM_CORES = 2
NUM_SUBCORES = 16
NUM_WORKERS = NUM_CORES * NUM_SUBCORES
NBLK = N_EDGES // EDGE_BLK
BLK_PER_W = -(-NBLK // NUM_WORKERS)
BLK_LOOP = BLK_PER_W + (BLK_PER_W % 2)
N_PAD = 10240
ROWS_PER_SUB = N_PAD // NUM_SUBCORES
WCHUNK = ROWS_PER_SUB // EDGE_BLK


def _sc_segment_sum(edge_attr, src, zeros_big, zeros_hist):
    mesh = plsc.VectorSubcoreMesh(core_axis_name="c", subcore_axis_name="s")
    cp = pltpu.CompilerParams()
    if "needs_layout_passes" in pltpu.CompilerParams.__dataclass_fields__:
        cp = dataclasses.replace(cp, needs_layout_passes=False)

    @functools.partial(
        pl.kernel,
        compiler_params=cp,
        out_type=[
            jax.ShapeDtypeStruct((NUM_CORES * N_PAD, DIM), jnp.float32),
            jax.ShapeDtypeStruct((NUM_WORKERS * N_PAD,), jnp.float32),
        ],
        mesh=mesh,
        scratch_types=[
            pltpu.VMEM_SHARED((N_PAD, DIM), jnp.float32),
            pltpu.VMEM((EDGE_BLK,), jnp.int32),
            pltpu.VMEM((EDGE_BLK,), jnp.int32),
            pltpu.VMEM((EDGE_BLK, DIM), jnp.float32),
            pltpu.VMEM((EDGE_BLK, DIM), jnp.float32),
            pltpu.VMEM((N_PAD,), jnp.float32),
            pltpu.SemaphoreType.DMA,
            pltpu.SemaphoreType.DMA,
            pltpu.SemaphoreType.DMA,
            pltpu.SemaphoreType.DMA,
        ],
    )
    def k(attr_hbm, src_hbm, zb_hbm, zh_hbm, sums_hbm, cnt_hbm,
          sh_sums, idx_v0, idx_v1, attr_v0, attr_v1, hist_v, sem0, sem1,
          ssem0, ssem1):
        cid = lax.axis_index("c")
        sid = lax.axis_index("s")
        wid = sid * NUM_CORES + cid

        row0 = sid * ROWS_PER_SUB
        pltpu.sync_copy(zb_hbm.at[pl.ds(0, EDGE_BLK)], attr_v0)
        pltpu.sync_copy(zh_hbm, hist_v)
        for j in range(WCHUNK):
            pltpu.sync_copy(attr_v0,
                            sh_sums.at[pl.ds(row0 + j * EDGE_BLK, EDGE_BLK)])
        plsc.subcore_barrier()

        ones16 = jnp.full((16,), 1.0, jnp.float32)
        bufs = ((idx_v0, attr_v0, sem0, ssem0), (idx_v1, attr_v1, sem1, ssem1))

        def loads(blk, buf):
            idxb, attrb, sem = buf[:3]
            base = blk * EDGE_BLK
            i_d = pltpu.make_async_copy(src_hbm.at[pl.ds(base, EDGE_BLK)],
                                        idxb, sem)
            a_d = pltpu.make_async_copy(attr_hbm.at[pl.ds(base, EDGE_BLK)],
                                        attrb, sem)
            return i_d, a_d

        def start_loads(blk, buf):
            i_d, a_d = loads(blk, buf)
            i_d.start()
            a_d.start()

        def wait_loads(blk, buf):
            i_d, a_d = loads(blk, buf)
            i_d.wait()
            a_d.wait()

        def scatter_desc(buf):
            idxb, attrb, _, ssem = buf
            return pltpu.make_async_copy(attrb, sh_sums.at[idxb], ssem)

        start_loads(wid, bufs[0])

        @pl.loop(0, BLK_LOOP, step=2)
        def _(kk):
            for b in range(2):
                cur = wid + (kk + b) * NUM_WORKERS
                nxt = cur + NUM_WORKERS
                buf = bufs[b]
                nbuf = bufs[1 - b]

                @pl.when(cur < NBLK)
                def _():
                    wait_loads(cur, buf)
                    idxb, attrb, _, ssem = buf
                    pltpu.async_copy(attrb, sh_sums.at[idxb], ssem, add=True)
                    for i in range(EDGE_BLK // 16):
                        idx16 = idxb[pl.ds(i * 16, 16)]
                        plsc.addupdate_scatter(hist_v, [idx16], ones16)

                    if b == 0:
                        @pl.when((nxt < NBLK) & (kk > 0))
                        def _():
                            scatter_desc(nbuf).wait()
                    else:
                        @pl.when(nxt < NBLK)
                        def _():
                            scatter_desc(nbuf).wait()

                    @pl.when(nxt < NBLK)
                    def _():
                        start_loads(nxt, nbuf)

        scatter_desc(bufs[0]).wait()
        scatter_desc(bufs[1]).wait()

        plsc.subcore_barrier()
        out0 = cid * N_PAD + row0
        for j in range(WCHUNK):
            pltpu.sync_copy(sh_sums.at[pl.ds(row0 + j * EDGE_BLK, EDGE_BLK)],
                            attr_v0)
            pltpu.sync_copy(attr_v0,
                            sums_hbm.at[pl.ds(out0 + j * EDGE_BLK, EDGE_BLK)])
        pltpu.sync_copy(hist_v, cnt_hbm.at[pl.ds(wid * N_PAD, N_PAD)])

    return k(edge_attr, src, zeros_big, zeros_hist)


BN = 1000


def _mlp_body(batch_ref, x_ref, s0_ref, s1_ref, cnt_ref, u_ref,
              w0x_ref, w0v_ref, w0u_ref, b0_ref, w1_ref, b1_ref,
              w2_ref, b2_ref, o_ref):
    f32 = jnp.float32
    cnt = jnp.sum(cnt_ref[...], axis=1)[:, None]
    v_e = (s0_ref[...] + s1_ref[...]) / jnp.maximum(cnt, 1.0)
    onehot = (batch_ref[...] ==
              lax.broadcasted_iota(jnp.int32, (1, N_GRAPHS), 1)).astype(f32)
    uw = lax.dot_general(u_ref[...], w0u_ref[...],
                         (((1,), (1,)), ((), ())), preferred_element_type=f32)
    t = lax.dot_general(x_ref[...], w0x_ref[...],
                        (((1,), (1,)), ((), ())), preferred_element_type=f32)
    t += lax.dot_general(v_e, w0v_ref[...],
                         (((1,), (1,)), ((), ())), preferred_element_type=f32)
    t += lax.dot_general(onehot, uw,
                         (((1,), (0,)), ((), ())), preferred_element_type=f32)
    h = jnp.maximum(t + b0_ref[...], 0.0)
    h = jnp.maximum(
        lax.dot_general(h, w1_ref[...], (((1,), (1,)), ((), ())),
                        preferred_element_type=f32) + b1_ref[...], 0.0)
    o_ref[...] = jnp.maximum(
        lax.dot_general(h, w2_ref[...], (((1,), (1,)), ((), ())),
                        preferred_element_type=f32) + b2_ref[...], 0.0)


def _tc_mlp(batch2d, x, s0, s1, cnt, u, w0x, w0v, w0u, b0, w1, b1, w2, b2):
    grid = (N_NODES // BN,)
    row = lambda i: (i, 0)
    full = lambda i: (0, 0)
    return pl.pallas_call(
        _mlp_body,
        grid=grid,
        in_specs=[
            pl.BlockSpec((BN, 1), row),
            pl.BlockSpec((BN, DIM), row),
            pl.BlockSpec((BN, DIM), row),
            pl.BlockSpec((BN, DIM), row),
            pl.BlockSpec((BN, NUM_WORKERS), lambda i: (i, 0)),
            pl.BlockSpec((N_GRAPHS, DIM), full),
            pl.BlockSpec((DIM, DIM), full),
            pl.BlockSpec((DIM, DIM), full),
            pl.BlockSpec((DIM, DIM), full),
            pl.BlockSpec((1, DIM), full),
            pl.BlockSpec((DIM, DIM), full),
            pl.BlockSpec((1, DIM), full),
            pl.BlockSpec((DIM, DIM), full),
            pl.BlockSpec((1, DIM), full),
        ],
        out_specs=pl.BlockSpec((BN, DIM), row),
        out_shape=jax.ShapeDtypeStruct((N_NODES, DIM), jnp.float32),
    )(batch2d, x, s0, s1, cnt, u, w0x, w0v, w0u, b0, w1, b1, w2, b2)


def kernel(x, edge_index, edge_attr, u, batch, W0, b0, W1, b1, W2, b2):
    src = edge_index[0, :]
    zeros_big = jnp.zeros((N_PAD, DIM), jnp.float32)
    zeros_hist = jnp.zeros((N_PAD,), jnp.float32)
    sums_p, cnt_p = _sc_segment_sum(edge_attr, src, zeros_big, zeros_hist)
    sums_p = sums_p.reshape(NUM_CORES, N_PAD, DIM)
    cnt_p = cnt_p.reshape(NUM_WORKERS, N_PAD)
    out = _tc_mlp(
        batch.reshape(N_NODES, 1), x,
        sums_p[0, :N_NODES], sums_p[1, :N_NODES],
        cnt_p[:, :N_NODES].T, u,
        W0[:, :DIM], W0[:, DIM:2 * DIM], W0[:, 2 * DIM:],
        b0.reshape(1, DIM), W1, b1.reshape(1, DIM), W2, b2.reshape(1, DIM))
    return out


# speedup vs baseline: 1.0435x; 1.0435x over previous
import dataclasses
import functools

import jax
import jax.numpy as jnp
from jax import lax
from jax.experimental import pallas as pl
from jax.experimental.pallas import tpu as pltpu
from jax.experimental.pallas import tpu_sc as plsc

N_NODES = 10000
N_EDGES = 320000
DIM = 128
N_GRAPHS = 64
EDGE_BLK = 128
NUM_CORES = 2
NUM_SUBCORES = 16
NUM_WORKERS = NUM_CORES * NUM_SUBCORES
NBLK = N_EDGES // EDGE_BLK
BLK_PER_W = -(-NBLK // NUM_WORKERS)
BLK_LOOP = BLK_PER_W + (BLK_PER_W % 2)
N_PAD = 10240
ROWS_PER_SUB = N_PAD // NUM_SUBCORES
WCHUNK = ROWS_PER_SUB // EDGE_BLK


def _sc_segment_sum(edge_attr, src, zeros_big, zeros_hist):
    mesh = plsc.VectorSubcoreMesh(core_axis_name="c", subcore_axis_name="s")
    cp = pltpu.CompilerParams()
    if "needs_layout_passes" in pltpu.CompilerParams.__dataclass_fields__:
        cp = dataclasses.replace(cp, needs_layout_passes=False)

    @functools.partial(
        pl.kernel,
        compiler_params=cp,
        out_type=[
            jax.ShapeDtypeStruct((NUM_CORES * N_PAD, DIM), jnp.float32),
            jax.ShapeDtypeStruct((NUM_WORKERS * N_PAD,), jnp.float32),
        ],
        mesh=mesh,
        scratch_types=[
            pltpu.VMEM_SHARED((N_PAD, DIM), jnp.float32),
            pltpu.VMEM((EDGE_BLK,), jnp.int32),
            pltpu.VMEM((EDGE_BLK,), jnp.int32),
            pltpu.VMEM((EDGE_BLK, DIM), jnp.float32),
            pltpu.VMEM((EDGE_BLK, DIM), jnp.float32),
            pltpu.VMEM((N_PAD,), jnp.float32),
            pltpu.SemaphoreType.DMA,
            pltpu.SemaphoreType.DMA,
            pltpu.SemaphoreType.DMA,
            pltpu.SemaphoreType.DMA,
        ],
    )
    def k(attr_hbm, src_hbm, zb_hbm, zh_hbm, sums_hbm, cnt_hbm,
          sh_sums, idx_v0, idx_v1, attr_v0, attr_v1, hist_v, sem0, sem1,
          ssem0, ssem1):
        cid = lax.axis_index("c")
        sid = lax.axis_index("s")
        wid = sid * NUM_CORES + cid

        row0 = sid * ROWS_PER_SUB
        pltpu.sync_copy(zb_hbm.at[pl.ds(0, EDGE_BLK)], attr_v0)
        pltpu.async_copy(zh_hbm, hist_v, ssem0)
        zfills = [pltpu.make_async_copy(
            attr_v0, sh_sums.at[pl.ds(row0 + j * EDGE_BLK, EDGE_BLK)], sem0)
            for j in range(WCHUNK)]
        for d in zfills:
            d.start()
        for d in zfills:
            d.wait()
        pltpu.make_async_copy(zh_hbm, hist_v, ssem0).wait()
        plsc.subcore_barrier()

        ones16 = jnp.full((16,), 1.0, jnp.float32)
        bufs = ((idx_v0, attr_v0, sem0, ssem0), (idx_v1, attr_v1, sem1, ssem1))

        def loads(blk, buf):
            idxb, attrb, sem = buf[:3]
            base = blk * EDGE_BLK
            i_d = pltpu.make_async_copy(src_hbm.at[pl.ds(base, EDGE_BLK)],
                                        idxb, sem)
            a_d = pltpu.make_async_copy(attr_hbm.at[pl.ds(base, EDGE_BLK)],
                                        attrb, sem)
            return i_d, a_d

        def start_loads(blk, buf):
            i_d, a_d = loads(blk, buf)
            i_d.start()
            a_d.start()

        def wait_loads(blk, buf):
            i_d, a_d = loads(blk, buf)
            i_d.wait()
            a_d.wait()

        def scatter_desc(buf):
            idxb, attrb, _, ssem = buf
            return pltpu.make_async_copy(attrb, sh_sums.at[idxb], ssem)

        start_loads(wid, bufs[0])

        @pl.loop(0, BLK_LOOP, step=2)
        def _(kk):
            for b in range(2):
                cur = wid + (kk + b) * NUM_WORKERS
                nxt = cur + NUM_WORKERS
                buf = bufs[b]
                nbuf = bufs[1 - b]

                @pl.when(cur < NBLK)
                def _():
                    wait_loads(cur, buf)

                    @pl.when(nxt < NBLK)
                    def _():
                        start_loads(nxt, nbuf)

                    idxb, attrb = buf[0], buf[1]
                    pltpu.sync_copy(attrb, sh_sums.at[idxb], add=True)
                    for i in range(EDGE_BLK // 16):
                        idx16 = idxb[pl.ds(i * 16, 16)]
                        plsc.addupdate_scatter(hist_v, [idx16], ones16)

        plsc.subcore_barrier()
        out0 = cid * N_PAD + row0
        pltpu.async_copy(hist_v, cnt_hbm.at[pl.ds(wid * N_PAD, N_PAD)], ssem0)
        wbufs = (attr_v0, attr_v1)
        wsems = (sem0, sem1)

        def rd(j, b):
            return pltpu.make_async_copy(
                sh_sums.at[pl.ds(row0 + j * EDGE_BLK, EDGE_BLK)],
                wbufs[b], wsems[b])

        def wr(j, b):
            return pltpu.make_async_copy(
                wbufs[b],
                sums_hbm.at[pl.ds(out0 + j * EDGE_BLK, EDGE_BLK)], wsems[b])

        rd(0, 0).start()
        for j in range(WCHUNK):
            b = j % 2
            nb = 1 - b
            rd(j, b).wait()
            if j + 1 < WCHUNK:
                if j >= 1:
                    wr(j - 1, nb).wait()
                rd(j + 1, nb).start()
            wr(j, b).start()
        wr(WCHUNK - 2, WCHUNK % 2).wait()
        wr(WCHUNK - 1, (WCHUNK - 1) % 2).wait()
        pltpu.make_async_copy(hist_v,
                              cnt_hbm.at[pl.ds(wid * N_PAD, N_PAD)],
                              ssem0).wait()

    return k(edge_attr, src, zeros_big, zeros_hist)


BN = 1000


def _mlp_body(batch_ref, x_ref, s0_ref, s1_ref, cnt_ref, u_ref,
              w0x_ref, w0v_ref, w0u_ref, b0_ref, w1_ref, b1_ref,
              w2_ref, b2_ref, o_ref):
    f32 = jnp.float32
    cnt = jnp.sum(cnt_ref[...], axis=1)[:, None]
    v_e = (s0_ref[...] + s1_ref[...]) / jnp.maximum(cnt, 1.0)
    onehot = (batch_ref[...] ==
              lax.broadcasted_iota(jnp.int32, (1, N_GRAPHS), 1)).astype(f32)
    uw = lax.dot_general(u_ref[...], w0u_ref[...],
                         (((1,), (1,)), ((), ())), preferred_element_type=f32)
    t = lax.dot_general(x_ref[...], w0x_ref[...],
                        (((1,), (1,)), ((), ())), preferred_element_type=f32)
    t += lax.dot_general(v_e, w0v_ref[...],
                         (((1,), (1,)), ((), ())), preferred_element_type=f32)
    t += lax.dot_general(onehot, uw,
                         (((1,), (0,)), ((), ())), preferred_element_type=f32)
    h = jnp.maximum(t + b0_ref[...], 0.0)
    h = jnp.maximum(
        lax.dot_general(h, w1_ref[...], (((1,), (1,)), ((), ())),
                        preferred_element_type=f32) + b1_ref[...], 0.0)
    o_ref[...] = jnp.maximum(
        lax.dot_general(h, w2_ref[...], (((1,), (1,)), ((), ())),
                        preferred_element_type=f32) + b2_ref[...], 0.0)


def _tc_mlp(batch2d, x, s0, s1, cnt, u, w0x, w0v, w0u, b0, w1, b1, w2, b2):
    grid = (N_NODES // BN,)
    row = lambda i: (i, 0)
    full = lambda i: (0, 0)
    return pl.pallas_call(
        _mlp_body,
        grid=grid,
        in_specs=[
            pl.BlockSpec((BN, 1), row),
            pl.BlockSpec((BN, DIM), row),
            pl.BlockSpec((BN, DIM), row),
            pl.BlockSpec((BN, DIM), row),
            pl.BlockSpec((BN, NUM_WORKERS), lambda i: (i, 0)),
            pl.BlockSpec((N_GRAPHS, DIM), full),
            pl.BlockSpec((DIM, DIM), full),
            pl.BlockSpec((DIM, DIM), full),
            pl.BlockSpec((DIM, DIM), full),
            pl.BlockSpec((1, DIM), full),
            pl.BlockSpec((DIM, DIM), full),
            pl.BlockSpec((1, DIM), full),
            pl.BlockSpec((DIM, DIM), full),
            pl.BlockSpec((1, DIM), full),
        ],
        out_specs=pl.BlockSpec((BN, DIM), row),
        out_shape=jax.ShapeDtypeStruct((N_NODES, DIM), jnp.float32),
    )(batch2d, x, s0, s1, cnt, u, w0x, w0v, w0u, b0, w1, b1, w2, b2)


def kernel(x, edge_index, edge_attr, u, batch, W0, b0, W1, b1, W2, b2):
    src = edge_index[0, :]
    zeros_big = jnp.zeros((N_PAD, DIM), jnp.float32)
    zeros_hist = jnp.zeros((N_PAD,), jnp.float32)
    sums_p, cnt_p = _sc_segment_sum(edge_attr, src, zeros_big, zeros_hist)
    sums_p = sums_p.reshape(NUM_CORES, N_PAD, DIM)
    cnt_p = cnt_p.reshape(NUM_WORKERS, N_PAD)
    out = _tc_mlp(
        batch.reshape(N_NODES, 1), x,
        sums_p[0, :N_NODES], sums_p[1, :N_NODES],
        cnt_p[:, :N_NODES].T, u,
        W0[:, :DIM], W0[:, DIM:2 * DIM], W0[:, 2 * DIM:],
        b0.reshape(1, DIM), W1, b1.reshape(1, DIM), W2, b2.reshape(1, DIM))
    return out


# speedup vs baseline: 1.0466x; 1.0029x over previous
import dataclasses
import functools

import jax
import jax.numpy as jnp
from jax import lax
from jax.experimental import pallas as pl
from jax.experimental.pallas import tpu as pltpu
from jax.experimental.pallas import tpu_sc as plsc

N_NODES = 10000
N_EDGES = 320000
DIM = 128
N_GRAPHS = 64
EDGE_BLK = 128
NUM_CORES = 2
NUM_SUBCORES = 16
NUM_WORKERS = NUM_CORES * NUM_SUBCORES
NBLK = N_EDGES // EDGE_BLK
BLK_PER_W = -(-NBLK // NUM_WORKERS)
BLK_LOOP = BLK_PER_W + (BLK_PER_W % 2)
N_PAD = 10240
ROWS_PER_SUB = N_PAD // NUM_SUBCORES
WCHUNK = ROWS_PER_SUB // EDGE_BLK


def _sc_segment_sum(edge_attr, src, zeros_big, zeros_hist):
    mesh = plsc.VectorSubcoreMesh(core_axis_name="c", subcore_axis_name="s")
    cp = pltpu.CompilerParams()
    if "needs_layout_passes" in pltpu.CompilerParams.__dataclass_fields__:
        cp = dataclasses.replace(cp, needs_layout_passes=False)

    @functools.partial(
        pl.kernel,
        compiler_params=cp,
        out_type=[
            jax.ShapeDtypeStruct((NUM_CORES * N_PAD, DIM), jnp.float32),
            jax.ShapeDtypeStruct((NUM_WORKERS * N_PAD,), jnp.float32),
        ],
        mesh=mesh,
        scratch_types=[
            pltpu.VMEM_SHARED((N_PAD, DIM), jnp.float32),
            pltpu.VMEM((EDGE_BLK,), jnp.int32),
            pltpu.VMEM((EDGE_BLK,), jnp.int32),
            pltpu.VMEM((EDGE_BLK, DIM), jnp.float32),
            pltpu.VMEM((EDGE_BLK, DIM), jnp.float32),
            pltpu.VMEM((N_PAD,), jnp.float32),
            pltpu.SemaphoreType.DMA,
            pltpu.SemaphoreType.DMA,
            pltpu.SemaphoreType.DMA,
            pltpu.SemaphoreType.DMA,
        ],
    )
    def k(attr_hbm, src_hbm, zb_hbm, zh_hbm, sums_hbm, cnt_hbm,
          sh_sums, idx_v0, idx_v1, attr_v0, attr_v1, hist_v, sem0, sem1,
          ssem0, ssem1):
        cid = lax.axis_index("c")
        sid = lax.axis_index("s")
        wid = sid * NUM_CORES + cid

        row0 = sid * ROWS_PER_SUB
        pltpu.sync_copy(zb_hbm.at[pl.ds(0, EDGE_BLK)], attr_v0)
        pltpu.async_copy(zh_hbm, hist_v, ssem0)
        zfills = [pltpu.make_async_copy(
            attr_v0, sh_sums.at[pl.ds(row0 + j * EDGE_BLK, EDGE_BLK)], sem0)
            for j in range(WCHUNK)]
        for d in zfills:
            d.start()
        for d in zfills:
            d.wait()
        pltpu.make_async_copy(zh_hbm, hist_v, ssem0).wait()
        plsc.subcore_barrier()

        ones16 = jnp.full((16,), 1.0, jnp.float32)
        bufs = ((idx_v0, attr_v0, sem0, ssem0), (idx_v1, attr_v1, sem1, ssem1))

        def loads(blk, buf):
            idxb, attrb, sem = buf[:3]
            base = blk * EDGE_BLK
            i_d = pltpu.make_async_copy(src_hbm.at[pl.ds(base, EDGE_BLK)],
                                        idxb, sem)
            a_d = pltpu.make_async_copy(attr_hbm.at[pl.ds(base, EDGE_BLK)],
                                        attrb, sem)
            return i_d, a_d

        def start_loads(blk, buf):
            i_d, a_d = loads(blk, buf)
            i_d.start()
            a_d.start()

        def wait_loads(blk, buf):
            i_d, a_d = loads(blk, buf)
            i_d.wait()
            a_d.wait()

        def scatter_desc(buf):
            idxb, attrb, _, ssem = buf
            return pltpu.make_async_copy(attrb, sh_sums.at[idxb], ssem)

        start_loads(wid, bufs[0])

        def process(buf):
            idxb, attrb = buf[0], buf[1]
            pltpu.sync_copy(attrb, sh_sums.at[idxb], add=True)
            for i in range(EDGE_BLK // 16):
                idx16 = idxb[pl.ds(i * 16, 16)]
                plsc.addupdate_scatter(hist_v, [idx16], ones16)

        FULL_PER_W = NBLK // NUM_WORKERS

        @pl.loop(0, FULL_PER_W, step=2)
        def _(kk):
            for b in range(2):
                cur = wid + (kk + b) * NUM_WORKERS
                nxt = cur + NUM_WORKERS
                buf = bufs[b]
                nbuf = bufs[1 - b]
                wait_loads(cur, buf)

                @pl.when(nxt < NBLK)
                def _():
                    start_loads(nxt, nbuf)

                process(buf)

        for t in range(FULL_PER_W, BLK_PER_W):
            tail = wid + t * NUM_WORKERS

            @pl.when(tail < NBLK)
            def _():
                wait_loads(tail, bufs[t % 2])
                process(bufs[t % 2])

        plsc.subcore_barrier()
        out0 = cid * N_PAD + row0
        pltpu.async_copy(hist_v, cnt_hbm.at[pl.ds(wid * N_PAD, N_PAD)], ssem0)
        wbufs = (attr_v0, attr_v1)
        wsems = (sem0, sem1)

        def rd(j, b):
            return pltpu.make_async_copy(
                sh_sums.at[pl.ds(row0 + j * EDGE_BLK, EDGE_BLK)],
                wbufs[b], wsems[b])

        def wr(j, b):
            return pltpu.make_async_copy(
                wbufs[b],
                sums_hbm.at[pl.ds(out0 + j * EDGE_BLK, EDGE_BLK)], wsems[b])

        rd(0, 0).start()
        for j in range(WCHUNK):
            b = j % 2
            nb = 1 - b
            rd(j, b).wait()
            if j + 1 < WCHUNK:
                if j >= 1:
                    wr(j - 1, nb).wait()
                rd(j + 1, nb).start()
            wr(j, b).start()
        wr(WCHUNK - 2, WCHUNK % 2).wait()
        wr(WCHUNK - 1, (WCHUNK - 1) % 2).wait()
        pltpu.make_async_copy(hist_v,
                              cnt_hbm.at[pl.ds(wid * N_PAD, N_PAD)],
                              ssem0).wait()

    return k(edge_attr, src, zeros_big, zeros_hist)


BN = 1000


def _mlp_body(batch_ref, x_ref, s0_ref, s1_ref, cnt_ref, u_ref,
              w0x_ref, w0v_ref, w0u_ref, b0_ref, w1_ref, b1_ref,
              w2_ref, b2_ref, o_ref):
    f32 = jnp.float32
    cnt = jnp.sum(cnt_ref[...], axis=1)[:, None]
    v_e = (s0_ref[...] + s1_ref[...]) / jnp.maximum(cnt, 1.0)
    onehot = (batch_ref[...] ==
              lax.broadcasted_iota(jnp.int32, (1, N_GRAPHS), 1)).astype(f32)
    uw = lax.dot_general(u_ref[...], w0u_ref[...],
                         (((1,), (1,)), ((), ())), preferred_element_type=f32)
    t = lax.dot_general(x_ref[...], w0x_ref[...],
                        (((1,), (1,)), ((), ())), preferred_element_type=f32)
    t += lax.dot_general(v_e, w0v_ref[...],
                         (((1,), (1,)), ((), ())), preferred_element_type=f32)
    t += lax.dot_general(onehot, uw,
                         (((1,), (0,)), ((), ())), preferred_element_type=f32)
    h = jnp.maximum(t + b0_ref[...], 0.0)
    h = jnp.maximum(
        lax.dot_general(h, w1_ref[...], (((1,), (1,)), ((), ())),
                        preferred_element_type=f32) + b1_ref[...], 0.0)
    o_ref[...] = jnp.maximum(
        lax.dot_general(h, w2_ref[...], (((1,), (1,)), ((), ())),
                        preferred_element_type=f32) + b2_ref[...], 0.0)


def _tc_mlp(batch2d, x, s0, s1, cnt, u, w0x, w0v, w0u, b0, w1, b1, w2, b2):
    grid = (N_NODES // BN,)
    row = lambda i: (i, 0)
    full = lambda i: (0, 0)
    return pl.pallas_call(
        _mlp_body,
        grid=grid,
        in_specs=[
            pl.BlockSpec((BN, 1), row),
            pl.BlockSpec((BN, DIM), row),
            pl.BlockSpec((BN, DIM), row),
            pl.BlockSpec((BN, DIM), row),
            pl.BlockSpec((BN, NUM_WORKERS), lambda i: (i, 0)),
            pl.BlockSpec((N_GRAPHS, DIM), full),
            pl.BlockSpec((DIM, DIM), full),
            pl.BlockSpec((DIM, DIM), full),
            pl.BlockSpec((DIM, DIM), full),
            pl.BlockSpec((1, DIM), full),
            pl.BlockSpec((DIM, DIM), full),
            pl.BlockSpec((1, DIM), full),
            pl.BlockSpec((DIM, DIM), full),
            pl.BlockSpec((1, DIM), full),
        ],
        out_specs=pl.BlockSpec((BN, DIM), row),
        out_shape=jax.ShapeDtypeStruct((N_NODES, DIM), jnp.float32),
    )(batch2d, x, s0, s1, cnt, u, w0x, w0v, w0u, b0, w1, b1, w2, b2)


def kernel(x, edge_index, edge_attr, u, batch, W0, b0, W1, b1, W2, b2):
    src = edge_index[0, :]
    zeros_big = jnp.zeros((N_PAD, DIM), jnp.float32)
    zeros_hist = jnp.zeros((N_PAD,), jnp.float32)
    sums_p, cnt_p = _sc_segment_sum(edge_attr, src, zeros_big, zeros_hist)
    sums_p = sums_p.reshape(NUM_CORES, N_PAD, DIM)
    cnt_p = cnt_p.reshape(NUM_WORKERS, N_PAD)
    out = _tc_mlp(
        batch.reshape(N_NODES, 1), x,
        sums_p[0, :N_NODES], sums_p[1, :N_NODES],
        cnt_p[:, :N_NODES].T, u,
        W0[:, :DIM], W0[:, DIM:2 * DIM], W0[:, 2 * DIM:],
        b0.reshape(1, DIM), W1, b1.reshape(1, DIM), W2, b2.reshape(1, DIM))
    return out
